# baseline (speedup 1.0000x reference)
import jax
import jax.numpy as jnp
from jax import lax
from jax.experimental import pallas as pl
from jax.experimental.pallas import tpu as pltpu
from jax.experimental.pallas import tpu_sc as plsc

_S = 64.0
_M = 0.4

_NC = 2
_NS = 16
_NW = _NC * _NS
_CHR = 16
_NBUF = 3
_ROWS_SC = 99840
_CPW = 195
_RPW = _CHR * _CPW


def _sc_scale_t(lt, labels_i32):
    c, b = lt.shape

    def body(lt_ref, lab_ref, out_ref, lab_v, *scr):
        cid = lax.axis_index("c")
        sid = lax.axis_index("s")
        wid = sid * _NC + cid
        base_row = wid * _RPW
        pltpu.sync_copy(lab_ref, lab_v)

        ins = scr[0:_NBUF]
        outs = scr[_NBUF:2 * _NBUF]
        lsems = scr[2 * _NBUF:3 * _NBUF]
        ssems = scr[3 * _NBUF:4 * _NBUF]

        def src_slice(t):
            r0 = pl.multiple_of(base_row + t * _CHR, 8)
            return lt_ref.at[pl.ds(r0, _CHR), :]

        def dst_slice(t):
            r0 = pl.multiple_of(base_row + t * _CHR, 8)
            return out_ref.at[pl.ds(r0, _CHR), :]

        for bb in range(_NBUF):
            pltpu.async_copy(src_slice(bb), ins[bb], lsems[bb])

        def group(g, carry):
            for bb in range(_NBUF):
                t = g * _NBUF + bb
                ib, ob, ls, ss = ins[bb], outs[bb], lsems[bb], ssems[bb]
                pltpu.make_async_copy(src_slice(t), ib, ls).wait()

                @pl.when(t >= _NBUF)
                def _(ob=ob, ss=ss, t=t):
                    pltpu.make_async_copy(ob, dst_slice(t - _NBUF), ss).wait()

                row0 = base_row + t * _CHR
                splats = [
                    jnp.full((16,), row0 + r, jnp.int32) for r in range(_CHR)
                ]

                def colslice(j, c2, ib=ib, ob=ob, splats=splats):
                    sl = pl.ds(j * 16, 16)
                    labv = lab_v[sl]
                    for r in range(_CHR):
                        m = labv == splats[r]
                        ob[r, sl] = (ib[r, sl] - jnp.where(m, _M, 0.0)) * _S
                    return c2

                lax.fori_loop(0, b // 16, colslice, 0)

                pltpu.async_copy(ob, dst_slice(t), ss)

                @pl.when(t + _NBUF < _CPW)
                def _(ib=ib, ls=ls, t=t):
                    pltpu.async_copy(src_slice(t + _NBUF), ib, ls)

            return carry

        lax.fori_loop(0, _CPW // _NBUF, group, 0)
        for bb in range(_NBUF):
            pltpu.make_async_copy(
                outs[bb], dst_slice(_CPW - _NBUF + bb), ssems[bb]).wait()

    mesh = plsc.VectorSubcoreMesh(
        core_axis_name="c", subcore_axis_name="s",
        num_cores=_NC, num_subcores=_NS,
    )
    fn = pl.kernel(
        body,
        out_type=jax.ShapeDtypeStruct((c, b), jnp.float32),
        mesh=mesh,
        scratch_types=(
            [pltpu.VMEM((b,), jnp.int32)]
            + [pltpu.VMEM((_CHR, b), jnp.float32)] * (2 * _NBUF)
            + [pltpu.SemaphoreType.DMA] * (2 * _NBUF)
        ),
    )
    return fn(lt, labels_i32)


def _tail_body(alias_ref, labels_ref, x_ref, o_ref):
    del alias_ref
    br, b = x_ref.shape
    rows = _ROWS_SC + jax.lax.broadcasted_iota(jnp.int32, (br, b), 0)
    lab = labels_ref[...]
    x = x_ref[...]
    o_ref[...] = (x - jnp.where(rows == lab, _M, 0.0)) * _S


def _tc_tail_t(sc_out, lt, labels_row):
    c, b = lt.shape
    tail = c - _ROWS_SC
    jblk = _ROWS_SC // tail
    return pl.pallas_call(
        _tail_body,
        grid=(1,),
        in_specs=[
            pl.BlockSpec(memory_space=pltpu.HBM),
            pl.BlockSpec((1, b), lambda i: (0, 0)),
            pl.BlockSpec((tail, b), lambda i: (jblk, 0)),
        ],
        out_specs=pl.BlockSpec((tail, b), lambda i: (jblk, 0)),
        out_shape=jax.ShapeDtypeStruct((c, b), jnp.float32),
        input_output_aliases={0: 0},
    )(sc_out, labels_row, lt)


def kernel(logits, norms, labels):
    del norms
    b, c = logits.shape
    lt = logits.T
    labels_i32 = labels.astype(jnp.int32)
    sc_out = _sc_scale_t(lt, labels_i32)
    out_t = _tc_tail_t(sc_out, lt, labels_i32.reshape(1, b))
    return out_t.T


# speedup vs baseline: 1.6945x; 1.6945x over previous
import jax
import jax.numpy as jnp
from jax.experimental import pallas as pl

_S = 64.0
_M = 0.4

_BLOCK_R = 3200


def _body(labels_ref, x_ref, o_ref):
    i = pl.program_id(0)
    br, b = x_ref.shape
    rows = i * _BLOCK_R + jax.lax.broadcasted_iota(jnp.int32, (br, b), 0)
    lab = labels_ref[...]
    x = x_ref[...]
    o_ref[...] = (x - jnp.where(rows == lab, _M, 0.0)) * _S


def kernel(logits, norms, labels):
    del norms
    b, c = logits.shape
    lt = logits.T
    labels_row = labels.astype(jnp.int32).reshape(1, b)
    out_t = pl.pallas_call(
        _body,
        grid=(pl.cdiv(c, _BLOCK_R),),
        in_specs=[
            pl.BlockSpec((1, b), lambda i: (0, 0)),
            pl.BlockSpec((_BLOCK_R, b), lambda i: (i, 0)),
        ],
        out_specs=pl.BlockSpec((_BLOCK_R, b), lambda i: (i, 0)),
        out_shape=jax.ShapeDtypeStruct((c, b), jnp.float32),
    )(labels_row, lt)
    return out_t.T
